# baseline (speedup 1.0000x reference)
import functools

import jax
import jax.numpy as jnp
import numpy as np
from jax import lax
from jax.experimental import pallas as pl
from jax.experimental.pallas import tpu as pltpu
from jax.experimental.pallas import tpu_sc as plsc

NC = 2
NS = 16
LANES = 16
ROWS_PER_STREAM = 128
NBUF = 4
FAST_RATIO = 4.7


def _chunk_split(n, nodes_per_chunk):
    total = -(-n // nodes_per_chunk)
    slow = -(-total // (NS * (FAST_RATIO + 1.0)))
    slow = max(NBUF, int(-(-slow // NBUF)) * NBUF)
    fast = -(-(total - NS * slow) // NS)
    fast = max(NBUF, -(-fast // NBUF) * NBUF)
    return fast, slow


def _build_sc_call(deg, emb, nch_fast, nch_slow):
    nodes_per_chunk = ROWS_PER_STREAM // deg
    nch_max = max(nch_fast, nch_slow)
    ngroup_max = nch_max // NBUF
    n_pad = NS * (nch_fast + nch_slow) * nodes_per_chunk
    nvec = emb // LANES
    inv_deg = np.float32(1.0 / deg)
    i32 = np.int32

    mesh = plsc.VectorSubcoreMesh(
        core_axis_name="c", subcore_axis_name="s",
        num_cores=NC, num_subcores=NS)

    @functools.partial(
        pl.kernel,
        out_type=jax.ShapeDtypeStruct((n_pad, emb), jnp.float32),
        mesh=mesh,
        scratch_types=(
            [pltpu.VMEM((nch_max, ROWS_PER_STREAM), jnp.int32)]
            + [pltpu.VMEM((ROWS_PER_STREAM, emb), jnp.float32)] * NBUF
            + [pltpu.VMEM((nodes_per_chunk, emb), jnp.float32)] * NBUF
            + [pltpu.SemaphoreType.DMA] * (2 * NBUF)
        ),
    )
    def sc_call(idx_hbm, table_hbm, out_hbm, idx_v, *rest):
        bufs = rest[:NBUF]
        ostage = rest[NBUF:2 * NBUF]
        gsems = rest[2 * NBUF:3 * NBUF]
        osems = rest[3 * NBUF:]
        c = lax.axis_index("c")
        s = lax.axis_index("s")
        w = c * i32(NS) + s

        on_fast = c == i32(0)
        nchunk = jnp.where(on_fast, i32(nch_fast), i32(nch_slow))
        gid0 = jnp.where(on_fast, s * i32(nch_fast),
                         i32(NS * nch_fast) + s * i32(nch_slow))

        pltpu.sync_copy(idx_hbm.at[w], idx_v)

        def gather_start(j, b):
            pltpu.async_copy(table_hbm.at[idx_v.at[j]], bufs[b], gsems[b])

        def gather_wait(j, b):
            pltpu.make_async_copy(
                table_hbm.at[idx_v.at[j]], bufs[b], gsems[b]).wait()

        def out_base(j):
            return (gid0 + j) * i32(nodes_per_chunk)

        for b in range(NBUF):
            gather_start(jnp.int32(b), b)

        def group_body(g, carry):
            for b in range(NBUF):
                j = g * i32(NBUF) + i32(b)

                @pl.when(j < nchunk)
                def _(j=j, b=b):
                    gather_wait(j, b)
                    buf, stg = bufs[b], ostage[b]

                    @pl.when(j >= i32(NBUF))
                    def _():
                        pltpu.make_async_copy(
                            stg, out_hbm.at[pl.ds(i32(0), nodes_per_chunk)],
                            osems[b]).wait()

                    def node_body(n, cc):
                        base = n * i32(deg)
                        for v0 in range(0, nvec, 4):
                            sls = [pl.ds(v * LANES, LANES)
                                   for v in range(v0, v0 + 4)]
                            accs = [buf[base, sl] for sl in sls]
                            for d in range(1, deg):
                                row = base + i32(d)
                                for k in range(4):
                                    accs[k] = accs[k] + buf[row, sls[k]]
                            for k in range(4):
                                stg[n, sls[k]] = accs[k] * inv_deg
                        return cc

                    lax.fori_loop(i32(0), i32(nodes_per_chunk), node_body, 0)
                    pltpu.async_copy(
                        stg, out_hbm.at[pl.ds(out_base(j), nodes_per_chunk)],
                        osems[b])

                    @pl.when(j + i32(NBUF) < nchunk)
                    def _(j=j, b=b):
                        gather_start(j + i32(NBUF), b)
            return carry

        lax.fori_loop(i32(0), i32(ngroup_max), group_body, 0)

        for b in range(NBUF):
            pltpu.make_async_copy(
                ostage[b], out_hbm.at[pl.ds(i32(0), nodes_per_chunk)],
                osems[b]).wait()

    return sc_call


def kernel(nodes, to_neighs, u2e_weight):
    del nodes
    n, deg = to_neighs.shape
    emb = u2e_weight.shape[1]

    nodes_per_chunk = ROWS_PER_STREAM // deg
    nch_fast, nch_slow = _chunk_split(n, nodes_per_chunk)
    nch_max = max(nch_fast, nch_slow)
    total_chunks = NS * (nch_fast + nch_slow)
    n_pad = total_chunks * nodes_per_chunk

    gids = np.full((NC * NS, nch_max), total_chunks, np.int64)
    for s in range(NS):
        gids[s, :nch_fast] = s * nch_fast + np.arange(nch_fast)
        gids[NS + s, :nch_slow] = (NS * nch_fast + s * nch_slow
                                   + np.arange(nch_slow))

    with jax.enable_x64(False):
        idx = to_neighs.astype(jnp.int32).reshape(-1)
        idx = jnp.pad(idx, (0, n_pad * deg - n * deg))
        flat = jnp.concatenate(
            [idx.reshape(total_chunks, ROWS_PER_STREAM),
             jnp.zeros((1, ROWS_PER_STREAM), jnp.int32)])
        idx3 = jnp.take(flat, gids, axis=0)

        table = u2e_weight.astype(jnp.float32)
        sc_call = _build_sc_call(deg, emb, nch_fast, nch_slow)
        out = sc_call(idx3, table)
        return out[:n]


# speedup vs baseline: 1.0211x; 1.0211x over previous
import functools

import jax
import jax.numpy as jnp
import numpy as np
from jax import lax
from jax.experimental import pallas as pl
from jax.experimental.pallas import tpu as pltpu
from jax.experimental.pallas import tpu_sc as plsc

NCORES = 1
NS = 16
LANES = 16
ROWS_PER_STREAM = 128
NBUF = 4


def _build_sc_call(deg, emb, nchunk):
    nodes_per_chunk = ROWS_PER_STREAM // deg
    ngroup = nchunk // NBUF
    n_pad = NCORES * NS * nchunk * nodes_per_chunk
    nvec = emb // LANES
    inv_deg = np.float32(1.0 / deg)
    i32 = np.int32

    mesh = plsc.VectorSubcoreMesh(
        core_axis_name="c", subcore_axis_name="s",
        num_cores=NCORES, num_subcores=NS)

    @functools.partial(
        pl.kernel,
        out_type=jax.ShapeDtypeStruct((n_pad, emb), jnp.float32),
        mesh=mesh,
        scratch_types=(
            [pltpu.VMEM((nchunk, ROWS_PER_STREAM), jnp.int32)]
            + [pltpu.VMEM((ROWS_PER_STREAM, emb), jnp.float32)] * NBUF
            + [pltpu.VMEM((nodes_per_chunk, emb), jnp.float32)] * NBUF
            + [pltpu.SemaphoreType.DMA] * (2 * NBUF)
        ),
    )
    def sc_call(idx_hbm, table_hbm, out_hbm, idx_v, *rest):
        bufs = rest[:NBUF]
        ostage = rest[NBUF:2 * NBUF]
        gsems = rest[2 * NBUF:3 * NBUF]
        osems = rest[3 * NBUF:]
        c = lax.axis_index("c")
        s = lax.axis_index("s")
        w = c * i32(NS) + s
        gid0 = w * i32(nchunk)

        pltpu.sync_copy(idx_hbm.at[w], idx_v)

        def gather_start(j, b):
            pltpu.async_copy(table_hbm.at[idx_v.at[j]], bufs[b], gsems[b])

        def gather_wait(j, b):
            pltpu.make_async_copy(
                table_hbm.at[idx_v.at[j]], bufs[b], gsems[b]).wait()

        for b in range(NBUF):
            gather_start(jnp.int32(b), b)

        def group_body(g, carry):
            for b in range(NBUF):
                j = g * i32(NBUF) + i32(b)
                gather_wait(j, b)
                buf, stg = bufs[b], ostage[b]

                @pl.when(j >= i32(NBUF))
                def _(stg=stg, b=b):
                    pltpu.make_async_copy(
                        stg, out_hbm.at[pl.ds(i32(0), nodes_per_chunk)],
                        osems[b]).wait()

                def node_body(n, cc, buf=buf, stg=stg):
                    base = n * i32(deg)
                    for v0 in range(0, nvec, 4):
                        sls = [pl.ds(v * LANES, LANES)
                               for v in range(v0, v0 + 4)]
                        accs = [buf[base, sl] for sl in sls]
                        for d in range(1, deg):
                            row = base + i32(d)
                            for k in range(4):
                                accs[k] = accs[k] + buf[row, sls[k]]
                        for k in range(4):
                            stg[n, sls[k]] = accs[k] * inv_deg
                    return cc

                lax.fori_loop(i32(0), i32(nodes_per_chunk), node_body, 0)
                pltpu.async_copy(
                    stg,
                    out_hbm.at[pl.ds((gid0 + j) * i32(nodes_per_chunk),
                                     nodes_per_chunk)],
                    osems[b])

                @pl.when(j + i32(NBUF) < i32(nchunk))
                def _(j=j, b=b):
                    gather_start(j + i32(NBUF), b)
            return carry

        lax.fori_loop(i32(0), i32(ngroup), group_body, 0)

        for b in range(NBUF):
            pltpu.make_async_copy(
                ostage[b], out_hbm.at[pl.ds(i32(0), nodes_per_chunk)],
                osems[b]).wait()

    return sc_call


def kernel(nodes, to_neighs, u2e_weight):
    del nodes
    n, deg = to_neighs.shape
    emb = u2e_weight.shape[1]

    nodes_per_chunk = ROWS_PER_STREAM // deg
    nw = NCORES * NS
    quantum = nodes_per_chunk * NBUF
    npw = ((n + nw - 1) // nw + quantum - 1) // quantum * quantum
    nchunk = npw // nodes_per_chunk
    n_pad = nw * npw

    with jax.enable_x64(False):
        idx = to_neighs.astype(jnp.int32).reshape(-1)
        idx = jnp.pad(idx, (0, n_pad * deg - n * deg))
        idx3 = idx.reshape(nw, nchunk, ROWS_PER_STREAM)

        table = u2e_weight.astype(jnp.float32)
        sc_call = _build_sc_call(deg, emb, nchunk)
        out = sc_call(idx3, table)
        return out[:n]
